# baseline (speedup 1.0000x reference)
import numpy as np
import jax
import jax.numpy as jnp
from jax import lax
from jax.experimental import pallas as pl
from jax.experimental.pallas import tpu as pltpu
from jax.experimental.pallas import tpu_sc as plsc

B, N, F, D = 1024, 64, 8, 128
IDX_PER_BATCH = N * F
IDX_ROWS = IDX_PER_BATCH // 128
NW = 32
BATCHES_PER_W = B // NW
OUT_ROWS = N + 1
NODES_PER_STREAM = 128 // F


def _sc_body(x_hbm, tab_hbm, tok_hbm, out_hbm,
             idx_v, rows_v, ob_v, g_sems, i_sem, o_sem):
    c = lax.axis_index("c")
    s = lax.axis_index("s")
    wid = s * 2 + c
    first = wid * BATCHES_PER_W

    def g_desc(iu, j):
        return pltpu.make_async_copy(
            tab_hbm.at[idx_v.at[iu * IDX_ROWS + j]],
            rows_v.at[pl.ds(j * 128, 128)], g_sems.at[j])

    def i_desc(b, iu):
        return pltpu.make_async_copy(
            x_hbm.at[b], idx_v.at[pl.ds(iu * IDX_ROWS, IDX_ROWS)], i_sem)

    def o_desc(b, u):
        return pltpu.make_async_copy(
            ob_v.at[pl.ds(u * OUT_ROWS, OUT_ROWS)], out_hbm.at[b], o_sem)

    def reduce_stream(j, u):
        @pl.loop(0, NODES_PER_STREAM)
        def _(n):
            src = j * 128 + n * F
            dst = u * OUT_ROWS + j * NODES_PER_STREAM + 1 + n
            for cp in range(D // 16):
                sl = pl.ds(cp * 16, 16)
                r = [rows_v[src + f, sl] for f in range(F)]
                a = [r[0] + r[1], r[2] + r[3], r[4] + r[5], r[6] + r[7]]
                ob_v[dst, sl] = (a[0] + a[1]) + (a[2] + a[3])

    def do_batch(b, u, fb=False, sb=False):
        if not fb and not sb:
            o_desc(b - 2, u).wait()
        i_desc(jnp.minimum(b + 1, B - 1), 1 - u).start()
        for j in range(IDX_ROWS):
            g_desc(u, j).wait()
            reduce_stream(j, u)
            if j == 0:
                i_desc(b, 1 - u).wait()
            g_desc(1 - u, j).start()
        o_desc(b, u).start()

    pltpu.sync_copy(tok_hbm, ob_v.at[pl.ds(0, 1)])
    pltpu.sync_copy(tok_hbm, ob_v.at[pl.ds(OUT_ROWS, 1)])
    pltpu.sync_copy(x_hbm.at[first], idx_v.at[pl.ds(0, IDX_ROWS)])
    for j in range(IDX_ROWS):
        g_desc(0, j).start()

    do_batch(first, 0, fb=True)
    do_batch(first + 1, 1, sb=True)

    @pl.loop(2, BATCHES_PER_W, step=2)
    def _(t):
        do_batch(first + t, 0)
        do_batch(first + t + 1, 1)

    for j in range(IDX_ROWS):
        g_desc(0, j).wait()
    o_desc(first + BATCHES_PER_W - 2, 0).wait()
    o_desc(first + BATCHES_PER_W - 1, 1).wait()


@jax.jit
def _atom_feature_sc(x3d, atom_table, graph_token):
    mesh = plsc.VectorSubcoreMesh(core_axis_name="c", subcore_axis_name="s")
    kfn = pl.kernel(
        _sc_body,
        out_type=jax.ShapeDtypeStruct((B, OUT_ROWS, D), jnp.float32),
        mesh=mesh,
        scratch_types=[
            pltpu.VMEM((2 * IDX_ROWS, 128), jnp.int32),
            pltpu.VMEM((IDX_ROWS * 128, D), jnp.float32),
            pltpu.VMEM((2 * OUT_ROWS, D), jnp.float32),
            pltpu.SemaphoreType.DMA((IDX_ROWS,)),
            pltpu.SemaphoreType.DMA,
            pltpu.SemaphoreType.DMA,
        ],
    )
    return kfn(x3d, atom_table, graph_token)


def kernel(x, atom_table, graph_token):
    x3d = x.reshape(B, IDX_ROWS, 128).astype(jnp.int32)
    return _atom_feature_sc(x3d, atom_table,
                            graph_token.astype(jnp.float32))


# speedup vs baseline: 1.2621x; 1.2621x over previous
import numpy as np
import jax
import jax.numpy as jnp
from jax import lax
from jax.experimental import pallas as pl
from jax.experimental.pallas import tpu as pltpu
from jax.experimental.pallas import tpu_sc as plsc

B, N, F, D = 1024, 64, 8, 128
IDX_PER_BATCH = N * F
IDX_ROWS = IDX_PER_BATCH // 128
NW = 32
BATCHES_PER_W = B // NW
OUT_ROWS = N + 1
NODES_PER_STREAM = 128 // F


def _sc_body(x_hbm, tab_hbm, tok_hbm, out_hbm,
             idx_v, rows_v, ob_v, g_sems, i_sem, o_sem):
    c = lax.axis_index("c")
    s = lax.axis_index("s")
    wid = s * 2 + c
    first = wid * BATCHES_PER_W

    def g_desc(iu, j):
        return pltpu.make_async_copy(
            tab_hbm.at[idx_v.at[iu * IDX_ROWS + j]],
            rows_v.at[pl.ds(j * 128, 128)], g_sems.at[j])

    def i_desc(b, iu):
        return pltpu.make_async_copy(
            x_hbm.at[b], idx_v.at[pl.ds(iu * IDX_ROWS, IDX_ROWS)], i_sem)

    def o_desc(b, u):
        return pltpu.make_async_copy(
            ob_v.at[pl.ds(u * OUT_ROWS, OUT_ROWS)], out_hbm.at[b], o_sem)

    def reduce_stream(j, u):
        @plsc.parallel_loop(0, NODES_PER_STREAM, unroll=2)
        def _(n):
            src = j * 128 + n * F
            dst = u * OUT_ROWS + j * NODES_PER_STREAM + 1 + n
            for cp in range(D // 16):
                sl = pl.ds(cp * 16, 16)
                r = [rows_v[src + f, sl] for f in range(F)]
                a = [r[0] + r[1], r[2] + r[3], r[4] + r[5], r[6] + r[7]]
                ob_v[dst, sl] = (a[0] + a[1]) + (a[2] + a[3])

    def do_batch(b, u, fb=False, sb=False):
        if not fb and not sb:
            o_desc(b - 2, u).wait()
        i_desc(jnp.minimum(b + 1, B - 1), 1 - u).start()
        for j in range(IDX_ROWS):
            g_desc(u, j).wait()
            reduce_stream(j, u)
            if j == 0:
                i_desc(b, 1 - u).wait()
            g_desc(1 - u, j).start()
        o_desc(b, u).start()

    pltpu.sync_copy(tok_hbm, ob_v.at[pl.ds(0, 1)])
    pltpu.sync_copy(tok_hbm, ob_v.at[pl.ds(OUT_ROWS, 1)])
    pltpu.sync_copy(x_hbm.at[first], idx_v.at[pl.ds(0, IDX_ROWS)])
    for j in range(IDX_ROWS):
        g_desc(0, j).start()

    do_batch(first, 0, fb=True)
    do_batch(first + 1, 1, sb=True)

    @pl.loop(2, BATCHES_PER_W, step=2)
    def _(t):
        do_batch(first + t, 0)
        do_batch(first + t + 1, 1)

    for j in range(IDX_ROWS):
        g_desc(0, j).wait()
    o_desc(first + BATCHES_PER_W - 2, 0).wait()
    o_desc(first + BATCHES_PER_W - 1, 1).wait()


@jax.jit
def _atom_feature_sc(x3d, atom_table, graph_token):
    mesh = plsc.VectorSubcoreMesh(core_axis_name="c", subcore_axis_name="s")
    kfn = pl.kernel(
        _sc_body,
        out_type=jax.ShapeDtypeStruct((B, OUT_ROWS, D), jnp.float32),
        mesh=mesh,
        scratch_types=[
            pltpu.VMEM((2 * IDX_ROWS, 128), jnp.int32),
            pltpu.VMEM((IDX_ROWS * 128, D), jnp.float32),
            pltpu.VMEM((2 * OUT_ROWS, D), jnp.float32),
            pltpu.SemaphoreType.DMA((IDX_ROWS,)),
            pltpu.SemaphoreType.DMA,
            pltpu.SemaphoreType.DMA,
        ],
    )
    return kfn(x3d, atom_table, graph_token)


def kernel(x, atom_table, graph_token):
    x3d = x.reshape(B, IDX_ROWS, 128).astype(jnp.int32)
    return _atom_feature_sc(x3d, atom_table,
                            graph_token.astype(jnp.float32))
